# Initial kernel scaffold; baseline (speedup 1.0000x reference)
import functools

import jax
import jax.numpy as jnp
from jax import lax
from jax.experimental import pallas as pl
from jax.experimental.pallas import tpu as pltpu
from jax.experimental.pallas import tpu_sc as plsc

N = 100000
E = 1600000
RAW = 128
W = 32
DEPTH = 10
SKIP = 2
EPS = 1e-5

NC = 2
NS = 16
LANES = 16
HALF = N // NC
SH = HALF + LANES
ZR = SH // NS
OR = HALF // NS
K = 128
NCH = -(-E // (NS * K))
EP = NCH * K
E2 = NS * EP

RB = 2000
G = N // RB


def _seg_sum_sc(h, srcp, dstp, zeros):
    mesh = plsc.VectorSubcoreMesh(core_axis_name="c", subcore_axis_name="s")

    @functools.partial(
        pl.kernel,
        out_type=jax.ShapeDtypeStruct((N, W), jnp.float32),
        mesh=mesh,
        scratch_types=[
            pltpu.VMEM((K,), jnp.int32),
            pltpu.VMEM((K,), jnp.int32),
            pltpu.VMEM((K,), jnp.int32),
            pltpu.VMEM((K, W), jnp.float32),
            pltpu.VMEM_SHARED((SH, W), jnp.float32),
            pltpu.SemaphoreType.DMA,
        ],
    )
    def seg(h_hbm, src_hbm, dst_hbm, z_hbm, out_hbm, sidx, didx, lidx, rows,
            aggsh, sem):
        c = lax.axis_index("c")
        s = lax.axis_index("s")
        base = c * HALF
        pltpu.sync_copy(z_hbm.at[pl.ds(s * ZR, ZR)], aggsh.at[pl.ds(s * ZR, ZR)])
        plsc.subcore_barrier()

        def body(i, carry):
            off = s * EP + i * K
            pltpu.sync_copy(src_hbm.at[pl.ds(off, K)], sidx)
            pltpu.sync_copy(dst_hbm.at[pl.ds(off, K)], didx)
            pltpu.async_copy(h_hbm.at[sidx], rows, sem).wait()
            for j in range(K // LANES):
                d = didx[pl.ds(j * LANES, LANES)]
                li = d - base
                oob = (li < 0) | (li >= HALF)
                lidx[pl.ds(j * LANES, LANES)] = jnp.where(oob, HALF, li)
            pltpu.sync_copy(rows, aggsh.at[lidx], add=True)
            return carry

        lax.fori_loop(0, NCH, body, 0)
        plsc.subcore_barrier()
        pltpu.sync_copy(aggsh.at[pl.ds(s * OR, OR)],
                        out_hbm.at[pl.ds(base + s * OR, OR)])

    return seg(h, srcp, dstp, zeros)


def _lrelu(x):
    return jnp.where(x >= 0, x, 0.01 * x)


def _p1_one(a, wm, bv):
    din = a.shape[1]

    def body(a_ref, w_ref, b_ref, y_ref, s_ref):
        y = jnp.dot(a_ref[...], w_ref[...],
                    preferred_element_type=jnp.float32) + b_ref[...]
        y_ref[...] = y

        @pl.when(pl.program_id(0) == 0)
        def _():
            s_ref[...] = jnp.zeros_like(s_ref)

        s_ref[0:1, :] += jnp.sum(y, 0, keepdims=True)
        s_ref[1:2, :] += jnp.sum(y * y, 0, keepdims=True)

    return pl.pallas_call(
        body,
        grid=(G,),
        in_specs=[
            pl.BlockSpec((RB, din), lambda i: (i, 0)),
            pl.BlockSpec((din, W), lambda i: (0, 0)),
            pl.BlockSpec((1, W), lambda i: (0, 0)),
        ],
        out_specs=[
            pl.BlockSpec((RB, W), lambda i: (i, 0)),
            pl.BlockSpec((2, W), lambda i: (0, 0)),
        ],
        out_shape=[
            jax.ShapeDtypeStruct((N, W), jnp.float32),
            jax.ShapeDtypeStruct((2, W), jnp.float32),
        ],
    )(a, wm, bv.reshape(1, W))


def _p1_two(a, b, wa, wb, bv):
    def body(a_ref, b_ref, wa_ref, wb_ref, bv_ref, y_ref, s_ref):
        y = (jnp.dot(a_ref[...], wa_ref[...], preferred_element_type=jnp.float32)
             + jnp.dot(b_ref[...], wb_ref[...], preferred_element_type=jnp.float32)
             + bv_ref[...])
        y_ref[...] = y

        @pl.when(pl.program_id(0) == 0)
        def _():
            s_ref[...] = jnp.zeros_like(s_ref)

        s_ref[0:1, :] += jnp.sum(y, 0, keepdims=True)
        s_ref[1:2, :] += jnp.sum(y * y, 0, keepdims=True)

    return pl.pallas_call(
        body,
        grid=(G,),
        in_specs=[
            pl.BlockSpec((RB, W), lambda i: (i, 0)),
            pl.BlockSpec((RB, W), lambda i: (i, 0)),
            pl.BlockSpec((W, W), lambda i: (0, 0)),
            pl.BlockSpec((W, W), lambda i: (0, 0)),
            pl.BlockSpec((1, W), lambda i: (0, 0)),
        ],
        out_specs=[
            pl.BlockSpec((RB, W), lambda i: (i, 0)),
            pl.BlockSpec((2, W), lambda i: (0, 0)),
        ],
        out_shape=[
            jax.ShapeDtypeStruct((N, W), jnp.float32),
            jax.ShapeDtypeStruct((2, W), jnp.float32),
        ],
    )(a, b, wa, wb, bv.reshape(1, W))


def _p2(y, sums, g, bv, skip=None):
    has_skip = skip is not None

    def body(*refs):
        if has_skip:
            y_ref, s_ref, g_ref, b_ref, sk_ref, o_ref = refs
            sk = sk_ref[...]
        else:
            y_ref, s_ref, g_ref, b_ref, o_ref = refs
            sk = 0.0
        m = s_ref[0:1, :] / N
        var = s_ref[1:2, :] / N - m * m
        inv = lax.rsqrt(var + EPS)
        hh = (y_ref[...] - m) * inv * g_ref[...] + b_ref[...]
        o_ref[...] = _lrelu(hh) + sk

    in_specs = [
        pl.BlockSpec((RB, W), lambda i: (i, 0)),
        pl.BlockSpec((2, W), lambda i: (0, 0)),
        pl.BlockSpec((1, W), lambda i: (0, 0)),
        pl.BlockSpec((1, W), lambda i: (0, 0)),
    ]
    args = [y, sums, g.reshape(1, W), bv.reshape(1, W)]
    if has_skip:
        in_specs.append(pl.BlockSpec((RB, W), lambda i: (i, 0)))
        args.append(skip)

    return pl.pallas_call(
        body,
        grid=(G,),
        in_specs=in_specs,
        out_specs=pl.BlockSpec((RB, W), lambda i: (i, 0)),
        out_shape=jax.ShapeDtypeStruct((N, W), jnp.float32),
    )(*args)


def _mlp_head(h, mlp_Ws, mlp_bs, fin_W, fin_b):
    def body(h_ref, w0, b0, w1, b1, w2, b2, w3, b3, wf, bf, o_ref):
        t = h_ref[...]
        for w_ref, b_ref in ((w0, b0), (w1, b1), (w2, b2), (w3, b3)):
            t = jnp.dot(t, w_ref[...], preferred_element_type=jnp.float32)
            t = _lrelu(t + b_ref[...])
        z = jnp.dot(t, wf[...], preferred_element_type=jnp.float32) + bf[...]
        o_ref[...] = 1.0 / (1.0 + jnp.exp(-z))

    in_specs = [pl.BlockSpec((RB, W), lambda i: (i, 0))]
    args = [h]
    for wm, bm in zip(mlp_Ws, mlp_bs):
        in_specs.append(pl.BlockSpec(wm.shape, lambda i: (0, 0)))
        in_specs.append(pl.BlockSpec((1, wm.shape[1]), lambda i: (0, 0)))
        args.append(wm)
        args.append(bm.reshape(1, -1))
    in_specs.append(pl.BlockSpec((W, 1), lambda i: (0, 0)))
    in_specs.append(pl.BlockSpec((1, 1), lambda i: (0, 0)))
    args.append(fin_W)
    args.append(fin_b.reshape(1, 1))

    return pl.pallas_call(
        body,
        grid=(G,),
        in_specs=in_specs,
        out_specs=pl.BlockSpec((RB, 1), lambda i: (i, 0)),
        out_shape=jax.ShapeDtypeStruct((N, 1), jnp.float32),
    )(*args)


def kernel(x, col_e_idx, init_W0, init_b0, init_g0, init_be0, init_W1,
           init_b1, init_g1, init_be1, coll_Ws, coll_bs, coll_gs, coll_bes,
           mlp_Ws, mlp_bs, fin_W, fin_b):
    src = col_e_idx[0]
    dst = col_e_idx[1]
    pad = E2 - E
    srcp = jnp.concatenate([src, jnp.zeros((pad,), jnp.int32)])
    dstp = jnp.concatenate([dst, jnp.full((pad,), N, jnp.int32)])
    zeros = jnp.zeros((SH, W), jnp.float32)

    y, s = _p1_one(x, init_W0, init_b0)
    h = _p2(y, s, init_g0, init_be0)
    y, s = _p1_one(h, init_W1, init_b1)
    h = _p2(y, s, init_g1, init_be1)

    middle = [h]
    for i in range(DEPTH):
        agg = _seg_sum_sc(h, srcp, dstp, zeros)
        y, s = _p1_two(h, agg, coll_Ws[i][:W], coll_Ws[i][W:], coll_bs[i])
        skip = middle[i - SKIP] if i - SKIP >= 0 else None
        h = _p2(y, s, coll_gs[i], coll_bes[i], skip)
        middle.append(h)

    return _mlp_head(h, mlp_Ws, mlp_bs, fin_W, fin_b)


# speedup vs baseline: 3.7807x; 3.7807x over previous
import functools

import jax
import jax.numpy as jnp
from jax import lax
from jax.experimental import pallas as pl
from jax.experimental.pallas import tpu as pltpu
from jax.experimental.pallas import tpu_sc as plsc

N = 100000
E = 1600000
RAW = 128
W = 32
DEPTH = 10
SKIP = 2
EPS = 1e-5

NC = 2
NS = 16
LANES = 16
HALF = N // NC
SH = 50176
ZR = SH // NS
OB = 3128
OLAST = HALF - (NS - 1) * OB
K = 128
NCH = -(-E // (NS * K))
EP = NCH * K
E2 = NS * EP

RB = 2000
G = N // RB


def _seg_sum_sc(h, srcp, dstp, zeros):
    mesh = plsc.VectorSubcoreMesh(core_axis_name="c", subcore_axis_name="s")

    @functools.partial(
        pl.kernel,
        out_type=jax.ShapeDtypeStruct((N, W), jnp.float32),
        mesh=mesh,
        scratch_types=[
            pltpu.VMEM((K,), jnp.int32),
            pltpu.VMEM((K,), jnp.int32),
            pltpu.VMEM((K,), jnp.int32),
            pltpu.VMEM((K, W), jnp.float32),
            pltpu.VMEM_SHARED((SH, W), jnp.float32),
            pltpu.SemaphoreType.DMA,
        ],
        compiler_params=pltpu.CompilerParams(use_tc_tiling_on_sc=False),
    )
    def seg(h_hbm, src_hbm, dst_hbm, z_hbm, out_hbm, sidx, didx, lidx, rows,
            aggsh, sem):
        c = lax.axis_index("c")
        s = lax.axis_index("s")
        base = c * HALF
        z0 = pl.multiple_of(s * ZR, 8)
        pltpu.sync_copy(z_hbm.at[pl.ds(z0, ZR)], aggsh.at[pl.ds(z0, ZR)])
        plsc.subcore_barrier()

        def body(i, carry):
            off = pl.multiple_of(s * EP + i * K, 8)
            pltpu.sync_copy(src_hbm.at[pl.ds(off, K)], sidx)
            pltpu.sync_copy(dst_hbm.at[pl.ds(off, K)], didx)
            pltpu.async_copy(h_hbm.at[sidx], rows, sem).wait()
            for j in range(K // LANES):
                d = didx[pl.ds(j * LANES, LANES)]
                li = d - base
                oob = (li < 0) | (li >= HALF)
                lidx[pl.ds(j * LANES, LANES)] = jnp.where(oob, HALF, li)
            pltpu.sync_copy(rows, aggsh.at[lidx], add=True)
            return carry

        lax.fori_loop(0, NCH, body, 0)
        plsc.subcore_barrier()
        o0 = pl.multiple_of(s * OB, 8)

        @pl.when(s < NS - 1)
        def _():
            pltpu.sync_copy(aggsh.at[pl.ds(o0, OB)],
                            out_hbm.at[pl.ds(base + o0, OB)])

        @pl.when(s == NS - 1)
        def _():
            pltpu.sync_copy(aggsh.at[pl.ds((NS - 1) * OB, OLAST)],
                            out_hbm.at[pl.ds(base + (NS - 1) * OB, OLAST)])

    return seg(h, srcp, dstp, zeros)


def _lrelu(x):
    return jnp.where(x >= 0, x, 0.01 * x)


def _p1_one(a, wm, bv):
    din = a.shape[1]

    def body(a_ref, w_ref, b_ref, y_ref, s_ref):
        y = jnp.dot(a_ref[...], w_ref[...],
                    preferred_element_type=jnp.float32) + b_ref[...]
        y_ref[...] = y

        @pl.when(pl.program_id(0) == 0)
        def _():
            s_ref[...] = jnp.zeros_like(s_ref)

        s_ref[0:1, :] += jnp.sum(y, 0, keepdims=True)
        s_ref[1:2, :] += jnp.sum(y * y, 0, keepdims=True)

    return pl.pallas_call(
        body,
        grid=(G,),
        in_specs=[
            pl.BlockSpec((RB, din), lambda i: (i, 0)),
            pl.BlockSpec((din, W), lambda i: (0, 0)),
            pl.BlockSpec((1, W), lambda i: (0, 0)),
        ],
        out_specs=[
            pl.BlockSpec((RB, W), lambda i: (i, 0)),
            pl.BlockSpec((2, W), lambda i: (0, 0)),
        ],
        out_shape=[
            jax.ShapeDtypeStruct((N, W), jnp.float32),
            jax.ShapeDtypeStruct((2, W), jnp.float32),
        ],
    )(a, wm, bv.reshape(1, W))


def _p1_two(a, b, wa, wb, bv):
    def body(a_ref, b_ref, wa_ref, wb_ref, bv_ref, y_ref, s_ref):
        y = (jnp.dot(a_ref[...], wa_ref[...], preferred_element_type=jnp.float32)
             + jnp.dot(b_ref[...], wb_ref[...], preferred_element_type=jnp.float32)
             + bv_ref[...])
        y_ref[...] = y

        @pl.when(pl.program_id(0) == 0)
        def _():
            s_ref[...] = jnp.zeros_like(s_ref)

        s_ref[0:1, :] += jnp.sum(y, 0, keepdims=True)
        s_ref[1:2, :] += jnp.sum(y * y, 0, keepdims=True)

    return pl.pallas_call(
        body,
        grid=(G,),
        in_specs=[
            pl.BlockSpec((RB, W), lambda i: (i, 0)),
            pl.BlockSpec((RB, W), lambda i: (i, 0)),
            pl.BlockSpec((W, W), lambda i: (0, 0)),
            pl.BlockSpec((W, W), lambda i: (0, 0)),
            pl.BlockSpec((1, W), lambda i: (0, 0)),
        ],
        out_specs=[
            pl.BlockSpec((RB, W), lambda i: (i, 0)),
            pl.BlockSpec((2, W), lambda i: (0, 0)),
        ],
        out_shape=[
            jax.ShapeDtypeStruct((N, W), jnp.float32),
            jax.ShapeDtypeStruct((2, W), jnp.float32),
        ],
    )(a, b, wa, wb, bv.reshape(1, W))


def _p2(y, sums, g, bv, skip=None):
    has_skip = skip is not None

    def body(*refs):
        if has_skip:
            y_ref, s_ref, g_ref, b_ref, sk_ref, o_ref = refs
            sk = sk_ref[...]
        else:
            y_ref, s_ref, g_ref, b_ref, o_ref = refs
            sk = 0.0
        m = s_ref[0:1, :] / N
        var = s_ref[1:2, :] / N - m * m
        inv = lax.rsqrt(var + EPS)
        hh = (y_ref[...] - m) * inv * g_ref[...] + b_ref[...]
        o_ref[...] = _lrelu(hh) + sk

    in_specs = [
        pl.BlockSpec((RB, W), lambda i: (i, 0)),
        pl.BlockSpec((2, W), lambda i: (0, 0)),
        pl.BlockSpec((1, W), lambda i: (0, 0)),
        pl.BlockSpec((1, W), lambda i: (0, 0)),
    ]
    args = [y, sums, g.reshape(1, W), bv.reshape(1, W)]
    if has_skip:
        in_specs.append(pl.BlockSpec((RB, W), lambda i: (i, 0)))
        args.append(skip)

    return pl.pallas_call(
        body,
        grid=(G,),
        in_specs=in_specs,
        out_specs=pl.BlockSpec((RB, W), lambda i: (i, 0)),
        out_shape=jax.ShapeDtypeStruct((N, W), jnp.float32),
    )(*args)


def _mlp_head(h, mlp_Ws, mlp_bs, fin_W, fin_b):
    def body(h_ref, w0, b0, w1, b1, w2, b2, w3, b3, wf, bf, o_ref):
        t = h_ref[...]
        for w_ref, b_ref in ((w0, b0), (w1, b1), (w2, b2), (w3, b3)):
            t = jnp.dot(t, w_ref[...], preferred_element_type=jnp.float32)
            t = _lrelu(t + b_ref[...])
        z = jnp.dot(t, wf[...], preferred_element_type=jnp.float32) + bf[...]
        o_ref[...] = 1.0 / (1.0 + jnp.exp(-z))

    in_specs = [pl.BlockSpec((RB, W), lambda i: (i, 0))]
    args = [h]
    for wm, bm in zip(mlp_Ws, mlp_bs):
        in_specs.append(pl.BlockSpec(wm.shape, lambda i: (0, 0)))
        in_specs.append(pl.BlockSpec((1, wm.shape[1]), lambda i: (0, 0)))
        args.append(wm)
        args.append(bm.reshape(1, -1))
    in_specs.append(pl.BlockSpec((W, 1), lambda i: (0, 0)))
    in_specs.append(pl.BlockSpec((1, 1), lambda i: (0, 0)))
    args.append(fin_W)
    args.append(fin_b.reshape(1, 1))

    return pl.pallas_call(
        body,
        grid=(G,),
        in_specs=in_specs,
        out_specs=pl.BlockSpec((RB, 1), lambda i: (i, 0)),
        out_shape=jax.ShapeDtypeStruct((N, 1), jnp.float32),
    )(*args)


def kernel(x, col_e_idx, init_W0, init_b0, init_g0, init_be0, init_W1,
           init_b1, init_g1, init_be1, coll_Ws, coll_bs, coll_gs, coll_bes,
           mlp_Ws, mlp_bs, fin_W, fin_b):
    src = col_e_idx[0]
    dst = col_e_idx[1]
    pad = E2 - E
    srcp = jnp.concatenate([src, jnp.zeros((pad,), jnp.int32)])
    dstp = jnp.concatenate([dst, jnp.full((pad,), N, jnp.int32)])
    zeros = jnp.zeros((SH, W), jnp.float32)

    y, s = _p1_one(x, init_W0, init_b0)
    h = _p2(y, s, init_g0, init_be0)
    y, s = _p1_one(h, init_W1, init_b1)
    h = _p2(y, s, init_g1, init_be1)

    middle = [h]
    for i in range(DEPTH):
        agg = _seg_sum_sc(h, srcp, dstp, zeros)
        y, s = _p1_two(h, agg, coll_Ws[i][:W], coll_Ws[i][W:], coll_bs[i])
        skip = middle[i - SKIP] if i - SKIP >= 0 else None
        h = _p2(y, s, coll_gs[i], coll_bes[i], skip)
        middle.append(h)

    return _mlp_head(h, mlp_Ws, mlp_bs, fin_W, fin_b)


# speedup vs baseline: 5.2249x; 1.3820x over previous
import functools

import jax
import jax.numpy as jnp
from jax import lax
from jax.experimental import pallas as pl
from jax.experimental.pallas import tpu as pltpu
from jax.experimental.pallas import tpu_sc as plsc

N = 100000
E = 1600000
RAW = 128
W = 32
DEPTH = 10
SKIP = 2
EPS = 1e-5

NC = 2
NS = 16
LANES = 16
HALF = N // NC
SH = 50176
ZR = SH // NS
OB = 3128
OLAST = HALF - (NS - 1) * OB
K = 128
NB = 8
RD = 4
NCH = ((-(-E // (NS * K)) + NB - 1) // NB) * NB
EP = NCH * K
E2 = NS * EP
NCHT = E2 // K

RB = 2000
G = N // RB


def _seg_sum_sc(h, comb, zeros):
    mesh = plsc.VectorSubcoreMesh(core_axis_name="c", subcore_axis_name="s")

    @functools.partial(
        pl.kernel,
        out_type=jax.ShapeDtypeStruct((N, W), jnp.float32),
        mesh=mesh,
        scratch_types=[
            pltpu.VMEM((NB, 2, K), jnp.int32),
            pltpu.VMEM((RD, K, W), jnp.float32),
            pltpu.VMEM_SHARED((SH, W), jnp.float32),
        ] + [pltpu.SemaphoreType.DMA] * (3 * NB),
        compiler_params=pltpu.CompilerParams(use_tc_tiling_on_sc=False),
    )
    def seg(h_hbm, comb_hbm, z_hbm, out_hbm, cidx, rows, aggsh, *sems):
        semI = sems[0:NB]
        semG = sems[NB:2 * NB]
        semS = sems[2 * NB:3 * NB]
        c = lax.axis_index("c")
        s = lax.axis_index("s")
        base = c * HALF
        q0 = s * NCH
        z0 = pl.multiple_of(s * ZR, 8)
        pltpu.sync_copy(z_hbm.at[pl.ds(z0, ZR)], aggsh.at[pl.ds(z0, ZR)])
        pltpu.async_copy(comb_hbm.at[c, q0 + 0], cidx.at[0], semI[0])
        pltpu.async_copy(comb_hbm.at[c, q0 + 1], cidx.at[1], semI[1])
        plsc.subcore_barrier()

        @pl.loop(0, NCH, step=NB)
        def outer(i):
            for j in range(NB):
                b = j
                r = j % RD
                b1 = (j - 1) % NB
                r1 = (j - 1) % RD
                b4 = (j - 4) % NB
                b2 = (j + 2) % NB
                q = i + j
                pltpu.make_async_copy(comb_hbm.at[c, q0], cidx.at[b],
                                      semI[b]).wait()

                @pl.when(q >= 4)
                def _():
                    pltpu.make_async_copy(rows.at[r],
                                          aggsh.at[cidx.at[b4, 1]],
                                          semS[b4]).wait()

                pltpu.async_copy(h_hbm.at[cidx.at[b, 0]], rows.at[r], semG[b])

                @pl.when(q + 2 < NCH)
                def _():
                    pltpu.async_copy(comb_hbm.at[c, q0 + q + 2], cidx.at[b2],
                                     semI[b2])

                @pl.when(q >= 1)
                def _():
                    pltpu.make_async_copy(h_hbm.at[cidx.at[b1, 0]],
                                          rows.at[r1], semG[b1]).wait()
                    pltpu.async_copy(rows.at[r1], aggsh.at[cidx.at[b1, 1]],
                                     semS[b1], add=True)

        bl = (NCH - 1) % NB
        rl = (NCH - 1) % RD
        pltpu.make_async_copy(h_hbm.at[cidx.at[bl, 0]], rows.at[rl],
                              semG[bl]).wait()
        pltpu.async_copy(rows.at[rl], aggsh.at[cidx.at[bl, 1]], semS[bl],
                         add=True)
        for p in range(NCH - 4, NCH):
            pltpu.make_async_copy(rows.at[p % RD],
                                  aggsh.at[cidx.at[p % NB, 1]],
                                  semS[p % NB]).wait()
        plsc.subcore_barrier()
        o0 = pl.multiple_of(s * OB, 8)

        @pl.when(s < NS - 1)
        def _():
            pltpu.sync_copy(aggsh.at[pl.ds(o0, OB)],
                            out_hbm.at[pl.ds(base + o0, OB)])

        @pl.when(s == NS - 1)
        def _():
            pltpu.sync_copy(aggsh.at[pl.ds((NS - 1) * OB, OLAST)],
                            out_hbm.at[pl.ds(base + (NS - 1) * OB, OLAST)])

    return seg(h, comb, zeros)


def _lrelu(x):
    return jnp.where(x >= 0, x, 0.01 * x)


def _p1_one(a, wm, bv):
    din = a.shape[1]

    def body(a_ref, w_ref, b_ref, y_ref, s_ref):
        y = jnp.dot(a_ref[...], w_ref[...],
                    preferred_element_type=jnp.float32) + b_ref[...]
        y_ref[...] = y

        @pl.when(pl.program_id(0) == 0)
        def _():
            s_ref[...] = jnp.zeros_like(s_ref)

        s_ref[0:1, :] += jnp.sum(y, 0, keepdims=True)
        s_ref[1:2, :] += jnp.sum(y * y, 0, keepdims=True)

    return pl.pallas_call(
        body,
        grid=(G,),
        in_specs=[
            pl.BlockSpec((RB, din), lambda i: (i, 0)),
            pl.BlockSpec((din, W), lambda i: (0, 0)),
            pl.BlockSpec((1, W), lambda i: (0, 0)),
        ],
        out_specs=[
            pl.BlockSpec((RB, W), lambda i: (i, 0)),
            pl.BlockSpec((2, W), lambda i: (0, 0)),
        ],
        out_shape=[
            jax.ShapeDtypeStruct((N, W), jnp.float32),
            jax.ShapeDtypeStruct((2, W), jnp.float32),
        ],
    )(a, wm, bv.reshape(1, W))


def _p1_two(a, b, wa, wb, bv):
    def body(a_ref, b_ref, wa_ref, wb_ref, bv_ref, y_ref, s_ref):
        y = (jnp.dot(a_ref[...], wa_ref[...], preferred_element_type=jnp.float32)
             + jnp.dot(b_ref[...], wb_ref[...], preferred_element_type=jnp.float32)
             + bv_ref[...])
        y_ref[...] = y

        @pl.when(pl.program_id(0) == 0)
        def _():
            s_ref[...] = jnp.zeros_like(s_ref)

        s_ref[0:1, :] += jnp.sum(y, 0, keepdims=True)
        s_ref[1:2, :] += jnp.sum(y * y, 0, keepdims=True)

    return pl.pallas_call(
        body,
        grid=(G,),
        in_specs=[
            pl.BlockSpec((RB, W), lambda i: (i, 0)),
            pl.BlockSpec((RB, W), lambda i: (i, 0)),
            pl.BlockSpec((W, W), lambda i: (0, 0)),
            pl.BlockSpec((W, W), lambda i: (0, 0)),
            pl.BlockSpec((1, W), lambda i: (0, 0)),
        ],
        out_specs=[
            pl.BlockSpec((RB, W), lambda i: (i, 0)),
            pl.BlockSpec((2, W), lambda i: (0, 0)),
        ],
        out_shape=[
            jax.ShapeDtypeStruct((N, W), jnp.float32),
            jax.ShapeDtypeStruct((2, W), jnp.float32),
        ],
    )(a, b, wa, wb, bv.reshape(1, W))


def _p2(y, sums, g, bv, skip=None):
    has_skip = skip is not None

    def body(*refs):
        if has_skip:
            y_ref, s_ref, g_ref, b_ref, sk_ref, o_ref = refs
            sk = sk_ref[...]
        else:
            y_ref, s_ref, g_ref, b_ref, o_ref = refs
            sk = 0.0
        m = s_ref[0:1, :] / N
        var = s_ref[1:2, :] / N - m * m
        inv = lax.rsqrt(var + EPS)
        hh = (y_ref[...] - m) * inv * g_ref[...] + b_ref[...]
        o_ref[...] = _lrelu(hh) + sk

    in_specs = [
        pl.BlockSpec((RB, W), lambda i: (i, 0)),
        pl.BlockSpec((2, W), lambda i: (0, 0)),
        pl.BlockSpec((1, W), lambda i: (0, 0)),
        pl.BlockSpec((1, W), lambda i: (0, 0)),
    ]
    args = [y, sums, g.reshape(1, W), bv.reshape(1, W)]
    if has_skip:
        in_specs.append(pl.BlockSpec((RB, W), lambda i: (i, 0)))
        args.append(skip)

    return pl.pallas_call(
        body,
        grid=(G,),
        in_specs=in_specs,
        out_specs=pl.BlockSpec((RB, W), lambda i: (i, 0)),
        out_shape=jax.ShapeDtypeStruct((N, W), jnp.float32),
    )(*args)


def _mlp_head(h, mlp_Ws, mlp_bs, fin_W, fin_b):
    def body(h_ref, w0, b0, w1, b1, w2, b2, w3, b3, wf, bf, o_ref):
        t = h_ref[...]
        for w_ref, b_ref in ((w0, b0), (w1, b1), (w2, b2), (w3, b3)):
            t = jnp.dot(t, w_ref[...], preferred_element_type=jnp.float32)
            t = _lrelu(t + b_ref[...])
        z = jnp.dot(t, wf[...], preferred_element_type=jnp.float32) + bf[...]
        o_ref[...] = 1.0 / (1.0 + jnp.exp(-z))

    in_specs = [pl.BlockSpec((RB, W), lambda i: (i, 0))]
    args = [h]
    for wm, bm in zip(mlp_Ws, mlp_bs):
        in_specs.append(pl.BlockSpec(wm.shape, lambda i: (0, 0)))
        in_specs.append(pl.BlockSpec((1, wm.shape[1]), lambda i: (0, 0)))
        args.append(wm)
        args.append(bm.reshape(1, -1))
    in_specs.append(pl.BlockSpec((W, 1), lambda i: (0, 0)))
    in_specs.append(pl.BlockSpec((1, 1), lambda i: (0, 0)))
    args.append(fin_W)
    args.append(fin_b.reshape(1, 1))

    return pl.pallas_call(
        body,
        grid=(G,),
        in_specs=in_specs,
        out_specs=pl.BlockSpec((RB, 1), lambda i: (i, 0)),
        out_shape=jax.ShapeDtypeStruct((N, 1), jnp.float32),
    )(*args)


def kernel(x, col_e_idx, init_W0, init_b0, init_g0, init_be0, init_W1,
           init_b1, init_g1, init_be1, coll_Ws, coll_bs, coll_gs, coll_bes,
           mlp_Ws, mlp_bs, fin_W, fin_b):
    src = col_e_idx[0]
    dst = col_e_idx[1]
    pad = E2 - E
    srcp = jnp.concatenate([src, jnp.zeros((pad,), jnp.int32)])
    dstp = jnp.concatenate([dst, jnp.full((pad,), N, jnp.int32)])
    src2 = srcp.reshape(NCHT, K)
    combs = []
    for c in range(NC):
        li = dstp - c * HALF
        li = jnp.where((li < 0) | (li >= HALF), HALF, li)
        combs.append(jnp.stack([src2, li.reshape(NCHT, K)], axis=1))
    comb = jnp.stack(combs)
    zeros = jnp.zeros((SH, W), jnp.float32)

    y, s = _p1_one(x, init_W0, init_b0)
    h = _p2(y, s, init_g0, init_be0)
    y, s = _p1_one(h, init_W1, init_b1)
    h = _p2(y, s, init_g1, init_be1)

    middle = [h]
    for i in range(DEPTH):
        agg = _seg_sum_sc(h, comb, zeros)
        y, s = _p1_two(h, agg, coll_Ws[i][:W], coll_Ws[i][W:], coll_bs[i])
        skip = middle[i - SKIP] if i - SKIP >= 0 else None
        h = _p2(y, s, coll_gs[i], coll_bes[i], skip)
        middle.append(h)

    return _mlp_head(h, mlp_Ws, mlp_bs, fin_W, fin_b)


# speedup vs baseline: 10.1026x; 1.9336x over previous
import functools

import jax
import jax.numpy as jnp
from jax import lax
from jax.experimental import pallas as pl
from jax.experimental.pallas import tpu as pltpu
from jax.experimental.pallas import tpu_sc as plsc

N = 100000
E = 1600000
RAW = 128
W = 32
DEPTH = 10
SKIP = 2
EPS = 1e-5

NC = 2
NS = 16
LANES = 16
FH = W // NC
OB = 6256
OLAST = N - (NS - 1) * OB
K = 128
NB = 8
RD = 4
NCH = ((-(-E // (NS * K)) + NB - 1) // NB) * NB
EP = NCH * K
E2 = NS * EP
NCHT = E2 // K

RB = 2000
G = N // RB


def _seg_sum_sc(h, comb, zeros):
    mesh = plsc.VectorSubcoreMesh(core_axis_name="c", subcore_axis_name="s")

    @functools.partial(
        pl.kernel,
        out_type=jax.ShapeDtypeStruct((N, W), jnp.float32),
        mesh=mesh,
        scratch_types=[
            pltpu.VMEM((NB, 2, K), jnp.int32),
            pltpu.VMEM((RD, K, FH), jnp.float32),
            pltpu.VMEM_SHARED((N + 8, FH), jnp.float32),
        ] + [pltpu.SemaphoreType.DMA] * (3 * NB),
        compiler_params=pltpu.CompilerParams(use_tc_tiling_on_sc=False),
    )
    def seg(h_hbm, comb_hbm, z_hbm, out_hbm, cidx, rows, aggsh, *sems):
        semI = sems[0:NB]
        semG = sems[NB:2 * NB]
        semS = sems[2 * NB:3 * NB]
        c = lax.axis_index("c")
        s = lax.axis_index("s")
        cf = pl.multiple_of(c * FH, 8)
        q0 = s * NCH
        z0 = pl.multiple_of(s * OB, 8)

        @pl.when(s < NS - 1)
        def _():
            pltpu.sync_copy(z_hbm.at[pl.ds(z0, OB)], aggsh.at[pl.ds(z0, OB)])

        @pl.when(s == NS - 1)
        def _():
            pltpu.sync_copy(z_hbm.at[pl.ds((NS - 1) * OB, OLAST)],
                            aggsh.at[pl.ds((NS - 1) * OB, OLAST)])

        pltpu.async_copy(comb_hbm.at[q0 + 0], cidx.at[0], semI[0])
        pltpu.async_copy(comb_hbm.at[q0 + 1], cidx.at[1], semI[1])
        plsc.subcore_barrier()

        @pl.loop(0, NCH, step=NB)
        def outer(i):
            for j in range(NB):
                b = j
                r = j % RD
                b1 = (j - 1) % NB
                r1 = (j - 1) % RD
                b4 = (j - 4) % NB
                b2 = (j + 2) % NB
                q = i + j
                pltpu.make_async_copy(comb_hbm.at[q0], cidx.at[b],
                                      semI[b]).wait()

                @pl.when(q >= 4)
                def _():
                    pltpu.make_async_copy(rows.at[r],
                                          aggsh.at[cidx.at[b4, 1]],
                                          semS[b4]).wait()

                pltpu.async_copy(h_hbm.at[c].at[cidx.at[b, 0]],
                                 rows.at[r], semG[b])

                @pl.when(q + 2 < NCH)
                def _():
                    pltpu.async_copy(comb_hbm.at[q0 + q + 2], cidx.at[b2],
                                     semI[b2])

                @pl.when(q >= 1)
                def _():
                    pltpu.make_async_copy(h_hbm.at[c].at[cidx.at[b1, 0]],
                                          rows.at[r1], semG[b1]).wait()
                    pltpu.async_copy(rows.at[r1], aggsh.at[cidx.at[b1, 1]],
                                     semS[b1], add=True)

        bl = (NCH - 1) % NB
        rl = (NCH - 1) % RD
        pltpu.make_async_copy(h_hbm.at[c].at[cidx.at[bl, 0]],
                              rows.at[rl], semG[bl]).wait()
        pltpu.async_copy(rows.at[rl], aggsh.at[cidx.at[bl, 1]], semS[bl],
                         add=True)
        for p in range(NCH - 4, NCH):
            pltpu.make_async_copy(rows.at[p % RD],
                                  aggsh.at[cidx.at[p % NB, 1]],
                                  semS[p % NB]).wait()
        plsc.subcore_barrier()
        o0 = pl.multiple_of(s * OB, 8)

        @pl.when(s < NS - 1)
        def _():
            pltpu.sync_copy(aggsh.at[pl.ds(o0, OB)],
                            out_hbm.at[pl.ds(o0, OB), pl.ds(cf, FH)])

        @pl.when(s == NS - 1)
        def _():
            pltpu.sync_copy(aggsh.at[pl.ds((NS - 1) * OB, OLAST)],
                            out_hbm.at[pl.ds((NS - 1) * OB, OLAST),
                                       pl.ds(cf, FH)])

    return seg(h, comb, zeros)


def _lrelu(x):
    return jnp.where(x >= 0, x, 0.01 * x)


def _p1_one(a, wm, bv):
    din = a.shape[1]

    def body(a_ref, w_ref, b_ref, y_ref, s_ref):
        y = jnp.dot(a_ref[...], w_ref[...],
                    preferred_element_type=jnp.float32) + b_ref[...]
        y_ref[...] = y

        @pl.when(pl.program_id(0) == 0)
        def _():
            s_ref[...] = jnp.zeros_like(s_ref)

        s_ref[0:1, :] += jnp.sum(y, 0, keepdims=True)
        s_ref[1:2, :] += jnp.sum(y * y, 0, keepdims=True)

    return pl.pallas_call(
        body,
        grid=(G,),
        in_specs=[
            pl.BlockSpec((RB, din), lambda i: (i, 0)),
            pl.BlockSpec((din, W), lambda i: (0, 0)),
            pl.BlockSpec((1, W), lambda i: (0, 0)),
        ],
        out_specs=[
            pl.BlockSpec((RB, W), lambda i: (i, 0)),
            pl.BlockSpec((2, W), lambda i: (0, 0)),
        ],
        out_shape=[
            jax.ShapeDtypeStruct((N, W), jnp.float32),
            jax.ShapeDtypeStruct((2, W), jnp.float32),
        ],
    )(a, wm, bv.reshape(1, W))


def _p1_two(a, b, wa, wb, bv):
    def body(a_ref, b_ref, wa_ref, wb_ref, bv_ref, y_ref, s_ref):
        y = (jnp.dot(a_ref[...], wa_ref[...], preferred_element_type=jnp.float32)
             + jnp.dot(b_ref[...], wb_ref[...], preferred_element_type=jnp.float32)
             + bv_ref[...])
        y_ref[...] = y

        @pl.when(pl.program_id(0) == 0)
        def _():
            s_ref[...] = jnp.zeros_like(s_ref)

        s_ref[0:1, :] += jnp.sum(y, 0, keepdims=True)
        s_ref[1:2, :] += jnp.sum(y * y, 0, keepdims=True)

    return pl.pallas_call(
        body,
        grid=(G,),
        in_specs=[
            pl.BlockSpec((RB, W), lambda i: (i, 0)),
            pl.BlockSpec((RB, W), lambda i: (i, 0)),
            pl.BlockSpec((W, W), lambda i: (0, 0)),
            pl.BlockSpec((W, W), lambda i: (0, 0)),
            pl.BlockSpec((1, W), lambda i: (0, 0)),
        ],
        out_specs=[
            pl.BlockSpec((RB, W), lambda i: (i, 0)),
            pl.BlockSpec((2, W), lambda i: (0, 0)),
        ],
        out_shape=[
            jax.ShapeDtypeStruct((N, W), jnp.float32),
            jax.ShapeDtypeStruct((2, W), jnp.float32),
        ],
    )(a, b, wa, wb, bv.reshape(1, W))


def _p2(y, sums, g, bv, skip=None, split=False):
    has_skip = skip is not None

    def body(*refs):
        if has_skip:
            y_ref, s_ref, g_ref, b_ref, sk_ref = refs[:5]
            out_refs = refs[5:]
            sk = sk_ref[...]
        else:
            y_ref, s_ref, g_ref, b_ref = refs[:4]
            out_refs = refs[4:]
            sk = 0.0
        m = s_ref[0:1, :] / N
        var = s_ref[1:2, :] / N - m * m
        inv = lax.rsqrt(var + EPS)
        hh = (y_ref[...] - m) * inv * g_ref[...] + b_ref[...]
        res = _lrelu(hh) + sk
        out_refs[0][...] = res
        if split:
            out_refs[1][0, :, :] = res[:, :FH]
            out_refs[1][1, :, :] = res[:, FH:]

    in_specs = [
        pl.BlockSpec((RB, W), lambda i: (i, 0)),
        pl.BlockSpec((2, W), lambda i: (0, 0)),
        pl.BlockSpec((1, W), lambda i: (0, 0)),
        pl.BlockSpec((1, W), lambda i: (0, 0)),
    ]
    args = [y, sums, g.reshape(1, W), bv.reshape(1, W)]
    if has_skip:
        in_specs.append(pl.BlockSpec((RB, W), lambda i: (i, 0)))
        args.append(skip)

    out_specs = [pl.BlockSpec((RB, W), lambda i: (i, 0))]
    out_shape = [jax.ShapeDtypeStruct((N, W), jnp.float32)]
    if split:
        out_specs.append(pl.BlockSpec((2, RB, FH), lambda i: (0, i, 0)))
        out_shape.append(jax.ShapeDtypeStruct((2, N, FH), jnp.float32))

    res = pl.pallas_call(
        body,
        grid=(G,),
        in_specs=in_specs,
        out_specs=out_specs,
        out_shape=out_shape,
    )(*args)
    return res if split else res[0]


def _mlp_head(h, mlp_Ws, mlp_bs, fin_W, fin_b):
    def body(h_ref, w0, b0, w1, b1, w2, b2, w3, b3, wf, bf, o_ref):
        t = h_ref[...]
        for w_ref, b_ref in ((w0, b0), (w1, b1), (w2, b2), (w3, b3)):
            t = jnp.dot(t, w_ref[...], preferred_element_type=jnp.float32)
            t = _lrelu(t + b_ref[...])
        z = jnp.dot(t, wf[...], preferred_element_type=jnp.float32) + bf[...]
        o_ref[...] = 1.0 / (1.0 + jnp.exp(-z))

    in_specs = [pl.BlockSpec((RB, W), lambda i: (i, 0))]
    args = [h]
    for wm, bm in zip(mlp_Ws, mlp_bs):
        in_specs.append(pl.BlockSpec(wm.shape, lambda i: (0, 0)))
        in_specs.append(pl.BlockSpec((1, wm.shape[1]), lambda i: (0, 0)))
        args.append(wm)
        args.append(bm.reshape(1, -1))
    in_specs.append(pl.BlockSpec((W, 1), lambda i: (0, 0)))
    in_specs.append(pl.BlockSpec((1, 1), lambda i: (0, 0)))
    args.append(fin_W)
    args.append(fin_b.reshape(1, 1))

    return pl.pallas_call(
        body,
        grid=(G,),
        in_specs=in_specs,
        out_specs=pl.BlockSpec((RB, 1), lambda i: (i, 0)),
        out_shape=jax.ShapeDtypeStruct((N, 1), jnp.float32),
    )(*args)


def kernel(x, col_e_idx, init_W0, init_b0, init_g0, init_be0, init_W1,
           init_b1, init_g1, init_be1, coll_Ws, coll_bs, coll_gs, coll_bes,
           mlp_Ws, mlp_bs, fin_W, fin_b):
    src = col_e_idx[0]
    dst = col_e_idx[1]
    pad = E2 - E
    srcp = jnp.concatenate([src, jnp.zeros((pad,), jnp.int32)])
    dstp = jnp.concatenate([dst, jnp.full((pad,), N, jnp.int32)])
    comb = jnp.stack([srcp.reshape(NCHT, K), dstp.reshape(NCHT, K)], axis=1)
    zeros = jnp.zeros((N, FH), jnp.float32)

    y, s = _p1_one(x, init_W0, init_b0)
    h = _p2(y, s, init_g0, init_be0)
    y, s = _p1_one(h, init_W1, init_b1)
    h, hs = _p2(y, s, init_g1, init_be1, split=True)

    middle = [h]
    for i in range(DEPTH):
        agg = _seg_sum_sc(hs, comb, zeros)
        y, s = _p1_two(h, agg, coll_Ws[i][:W], coll_Ws[i][W:], coll_bs[i])
        skip = middle[i - SKIP] if i - SKIP >= 0 else None
        need_split = i < DEPTH - 1
        res = _p2(y, s, coll_gs[i], coll_bes[i], skip, split=need_split)
        if need_split:
            h, hs = res
        else:
            h = res
        middle.append(h)

    return _mlp_head(h, mlp_Ws, mlp_bs, fin_W, fin_b)
